# baseline (speedup 1.0000x reference)
import jax
import jax.numpy as jnp
from jax.experimental import pallas as pl
from jax.experimental.pallas import tpu as pltpu

N = 4096
H = 512
D = 128
LEAKY = 0.2
B = 1024
NB = N // B
KC = 1024
NK = N // KC
C = 2048
NC = N // C
EPS = 1e-5


def _ln(h, w, b):
    ones_dd = jnp.ones((D, D), jnp.bfloat16)
    hb = h.astype(jnp.bfloat16)
    mu = jnp.dot(hb, ones_dd, preferred_element_type=jnp.float32) * (1.0 / D)
    hh = (h * h).astype(jnp.bfloat16)
    m2 = jnp.dot(hh, ones_dd, preferred_element_type=jnp.float32) * (1.0 / D)
    var = m2 - mu * mu
    return (h - mu) * jax.lax.rsqrt(var + EPS) * w + b


def _body(att_ref, sp_ref, embs_ref, ln0w_ref, ln0b_ref, ln1w_ref, ln1b_ref,
          out_ref, adj_scr, t0_scr, t1_scr):
    p = pl.program_id(0)
    i = pl.program_id(1)

    @pl.when(p == 0)
    def _phase0():
        rows = pl.ds(i * B, B)
        adj_blk = jax.lax.dot_general(
            att_ref[...].astype(jnp.float8_e4m3fn), sp_ref[...],
            (((1,), (0,)), ((), ())),
            preferred_element_type=jnp.float32)
        adj_bf = adj_blk.astype(jnp.bfloat16)
        adj_scr[rows, :] = adj_bf

        @pl.when(i == 0)
        def _zero():
            t0_scr[...] = jnp.zeros_like(t0_scr)

        t0_scr[...] += jax.lax.dot_general(
            adj_bf, embs_ref[rows, :].astype(jnp.bfloat16),
            (((0,), (0,)), ((), ())),
            preferred_element_type=jnp.float32)

    @pl.when((p == 1) & (i < NC))
    def _phase1():
        rows = pl.ds(i * C, C)
        adj_blk = adj_scr[rows, :]
        h = jnp.dot(adj_blk, t0_scr[...].astype(jnp.bfloat16),
                    preferred_element_type=jnp.float32)
        h = jnp.where(h >= 0, h, LEAKY * h)
        e1 = _ln(h, ln0w_ref[...], ln0b_ref[...]) + embs_ref[rows, :]

        @pl.when(i == 0)
        def _zero():
            t1_scr[...] = jnp.zeros_like(t1_scr)

        t1_scr[...] += jax.lax.dot_general(
            adj_blk, e1.astype(jnp.bfloat16), (((0,), (0,)), ((), ())),
            preferred_element_type=jnp.float32)

    @pl.when((p == 2) & (i < NC))
    def _phase2():
        rows = pl.ds(i * C, C)
        adj_blk = adj_scr[rows, :]
        h = jnp.dot(adj_blk, t1_scr[...].astype(jnp.bfloat16),
                    preferred_element_type=jnp.float32)
        out_ref[rows, :] = (_ln(h, ln1w_ref[...], ln1b_ref[...])
                            + embs_ref[rows, :])


def kernel(embs, sparse_adj, att_adj, ln0_w, ln0_b, ln1_w, ln1_b):
    grid = (3, NB)
    out = pl.pallas_call(
        _body,
        grid=grid,
        in_specs=[
            pl.BlockSpec((B, N), lambda p, i: (jnp.where(p == 0, i, NB - 1), 0)),
            pl.BlockSpec((N, H), lambda p, i: (0, 0)),
            pl.BlockSpec((N, D), lambda p, i: (0, 0)),
            pl.BlockSpec((1, D), lambda p, i: (0, 0)),
            pl.BlockSpec((1, D), lambda p, i: (0, 0)),
            pl.BlockSpec((1, D), lambda p, i: (0, 0)),
            pl.BlockSpec((1, D), lambda p, i: (0, 0)),
        ],
        out_specs=pl.BlockSpec((N, D), lambda p, i: (0, 0)),
        out_shape=jax.ShapeDtypeStruct((N, D), jnp.float32),
        scratch_shapes=[
            pltpu.VMEM((N, H), jnp.bfloat16),
            pltpu.VMEM((H, D), jnp.float32),
            pltpu.VMEM((H, D), jnp.float32),
        ],
        compiler_params=pltpu.CompilerParams(
            dimension_semantics=("arbitrary", "arbitrary")),
    )(att_adj, sparse_adj.astype(jnp.float8_e4m3fn), embs,
      ln0_w.reshape(1, D), ln0_b.reshape(1, D),
      ln1_w.reshape(1, D), ln1_b.reshape(1, D))
    return out


# speedup vs baseline: 1.0568x; 1.0568x over previous
import jax
import jax.numpy as jnp
from jax.experimental import pallas as pl
from jax.experimental.pallas import tpu as pltpu

N = 4096
H = 512
D = 128
LEAKY = 0.2
B = 1024
NB = N // B
KC = 1024
NK = N // KC
C = 4096
NC = N // C
EPS = 1e-5


def _ln(h, w, b):
    mu = jnp.mean(h, axis=-1, keepdims=True)
    var = jnp.mean((h - mu) ** 2, axis=-1, keepdims=True)
    return (h - mu) * jax.lax.rsqrt(var + EPS) * w + b


def _body(att_ref, sp_ref, embs_ref, ln0w_ref, ln0b_ref, ln1w_ref, ln1b_ref,
          out_ref, adj_scr, t0_scr, t1_scr):
    p = pl.program_id(0)
    i = pl.program_id(1)

    @pl.when(p == 0)
    def _phase0():
        rows = pl.ds(i * B, B)
        adj_blk = jax.lax.dot_general(
            att_ref[...].astype(jnp.float8_e4m3fn), sp_ref[...],
            (((1,), (0,)), ((), ())),
            preferred_element_type=jnp.float32)
        adj_bf = adj_blk.astype(jnp.bfloat16)
        adj_scr[rows, :] = adj_bf

        @pl.when(i == 0)
        def _zero():
            t0_scr[...] = jnp.zeros_like(t0_scr)

        t0_scr[...] += jax.lax.dot_general(
            adj_bf, embs_ref[rows, :].astype(jnp.bfloat16),
            (((0,), (0,)), ((), ())),
            preferred_element_type=jnp.float32)

    @pl.when((p == 1) & (i < NC))
    def _phase1():
        rows = pl.ds(i * C, C)
        adj_blk = adj_scr[rows, :]
        h = jnp.dot(adj_blk, t0_scr[...].astype(jnp.bfloat16),
                    preferred_element_type=jnp.float32)
        h = jnp.where(h >= 0, h, LEAKY * h)
        e1 = _ln(h, ln0w_ref[...], ln0b_ref[...]) + embs_ref[rows, :]

        @pl.when(i == 0)
        def _zero():
            t1_scr[...] = jnp.zeros_like(t1_scr)

        t1_scr[...] += jax.lax.dot_general(
            adj_blk, e1.astype(jnp.bfloat16), (((0,), (0,)), ((), ())),
            preferred_element_type=jnp.float32)

    @pl.when((p == 2) & (i < NC))
    def _phase2():
        rows = pl.ds(i * C, C)
        adj_blk = adj_scr[rows, :]
        h = jnp.dot(adj_blk, t1_scr[...].astype(jnp.bfloat16),
                    preferred_element_type=jnp.float32)
        out_ref[rows, :] = (_ln(h, ln1w_ref[...], ln1b_ref[...])
                            + embs_ref[rows, :])


def kernel(embs, sparse_adj, att_adj, ln0_w, ln0_b, ln1_w, ln1_b):
    grid = (3, NB)
    out = pl.pallas_call(
        _body,
        grid=grid,
        in_specs=[
            pl.BlockSpec((B, N), lambda p, i: (jnp.where(p == 0, i, NB - 1), 0)),
            pl.BlockSpec((N, H), lambda p, i: (0, 0)),
            pl.BlockSpec((N, D), lambda p, i: (0, 0)),
            pl.BlockSpec((1, D), lambda p, i: (0, 0)),
            pl.BlockSpec((1, D), lambda p, i: (0, 0)),
            pl.BlockSpec((1, D), lambda p, i: (0, 0)),
            pl.BlockSpec((1, D), lambda p, i: (0, 0)),
        ],
        out_specs=pl.BlockSpec((N, D), lambda p, i: (0, 0)),
        out_shape=jax.ShapeDtypeStruct((N, D), jnp.float32),
        scratch_shapes=[
            pltpu.VMEM((N, H), jnp.bfloat16),
            pltpu.VMEM((H, D), jnp.float32),
            pltpu.VMEM((H, D), jnp.float32),
        ],
        compiler_params=pltpu.CompilerParams(
            dimension_semantics=("arbitrary", "arbitrary")),
    )(att_adj, sparse_adj.astype(jnp.float8_e4m3fn), embs,
      ln0_w.reshape(1, D), ln0_b.reshape(1, D),
      ln1_w.reshape(1, D), ln1_b.reshape(1, D))
    return out


# speedup vs baseline: 1.0715x; 1.0139x over previous
import jax
import jax.numpy as jnp
from jax.experimental import pallas as pl
from jax.experimental.pallas import tpu as pltpu

N = 4096
H = 512
D = 128
LEAKY = 0.2
B = 1024
NB = N // B
KC = 1024
NK = N // KC
C = 4096
NC = N // C
EPS = 1e-5


def _ln(h, w, b):
    mu = jnp.mean(h, axis=-1, keepdims=True)
    var = jnp.mean((h - mu) ** 2, axis=-1, keepdims=True)
    return (h - mu) * jax.lax.rsqrt(var + EPS) * w + b


def _body(att_ref, sp_ref, embs_ref, ln0w_ref, ln0b_ref, ln1w_ref, ln1b_ref,
          out_ref, adj_scr, t0_scr, t1_scr):
    s = pl.program_id(0)
    i = s

    @pl.when(s < NB)
    def _phase0():
        rows = pl.ds(i * B, B)
        adj_blk = jax.lax.dot_general(
            att_ref[...].astype(jnp.float8_e4m3fn), sp_ref[...],
            (((1,), (0,)), ((), ())),
            preferred_element_type=jnp.float32)
        adj_bf = adj_blk.astype(jnp.bfloat16)
        adj_scr[rows, :] = adj_bf

        @pl.when(i == 0)
        def _zero():
            t0_scr[...] = jnp.zeros_like(t0_scr)

        t0_scr[...] += jax.lax.dot_general(
            adj_bf, embs_ref[rows, :].astype(jnp.bfloat16),
            (((0,), (0,)), ((), ())),
            preferred_element_type=jnp.float32)

    @pl.when(s == NB)
    def _phase1():
        rows = pl.ds(0, C)
        adj_blk = adj_scr[rows, :]
        h = jnp.dot(adj_blk, t0_scr[...].astype(jnp.bfloat16),
                    preferred_element_type=jnp.float32)
        h = jnp.where(h >= 0, h, LEAKY * h)
        e1 = _ln(h, ln0w_ref[...], ln0b_ref[...]) + embs_ref[rows, :]

        t1_scr[...] = jnp.zeros_like(t1_scr)
        t1_scr[...] += jax.lax.dot_general(
            adj_blk, e1.astype(jnp.bfloat16), (((0,), (0,)), ((), ())),
            preferred_element_type=jnp.float32)

    @pl.when(s == NB + 1)
    def _phase2():
        rows = pl.ds(0, C)
        adj_blk = adj_scr[rows, :]
        h = jnp.dot(adj_blk, t1_scr[...].astype(jnp.bfloat16),
                    preferred_element_type=jnp.float32)
        out_ref[rows, :] = (_ln(h, ln1w_ref[...], ln1b_ref[...])
                            + embs_ref[rows, :])


def kernel(embs, sparse_adj, att_adj, ln0_w, ln0_b, ln1_w, ln1_b):
    grid = (NB + 2,)
    out = pl.pallas_call(
        _body,
        grid=grid,
        in_specs=[
            pl.BlockSpec((B, N), lambda s: (jnp.minimum(s, NB - 1), 0)),
            pl.BlockSpec((N, H), lambda s: (0, 0)),
            pl.BlockSpec((N, D), lambda s: (0, 0)),
            pl.BlockSpec((1, D), lambda s: (0, 0)),
            pl.BlockSpec((1, D), lambda s: (0, 0)),
            pl.BlockSpec((1, D), lambda s: (0, 0)),
            pl.BlockSpec((1, D), lambda s: (0, 0)),
        ],
        out_specs=pl.BlockSpec((N, D), lambda s: (0, 0)),
        out_shape=jax.ShapeDtypeStruct((N, D), jnp.float32),
        scratch_shapes=[
            pltpu.VMEM((N, H), jnp.bfloat16),
            pltpu.VMEM((H, D), jnp.float32),
            pltpu.VMEM((H, D), jnp.float32),
        ],
        compiler_params=pltpu.CompilerParams(
            dimension_semantics=("arbitrary",)),
    )(att_adj, sparse_adj.astype(jnp.float8_e4m3fn), embs,
      ln0_w.reshape(1, D), ln0_b.reshape(1, D),
      ln1_w.reshape(1, D), ln1_b.reshape(1, D))
    return out


# speedup vs baseline: 1.1028x; 1.0292x over previous
import jax
import jax.numpy as jnp
from jax.experimental import pallas as pl
from jax.experimental.pallas import tpu as pltpu

N = 4096
H = 512
D = 128
LEAKY = 0.2
B = 512
NB = N // B
KC = 1024
NK = N // KC
C = 4096
NC = N // C
EPS = 1e-5


def _ln(h, w, b):
    mu = jnp.mean(h, axis=-1, keepdims=True)
    var = jnp.mean((h - mu) ** 2, axis=-1, keepdims=True)
    return (h - mu) * jax.lax.rsqrt(var + EPS) * w + b


def _body(att_ref, sp_ref, embs_ref, ln0w_ref, ln0b_ref, ln1w_ref, ln1b_ref,
          out_ref, adj_scr, t0_scr, t1_scr):
    s = pl.program_id(0)
    i = s

    @pl.when(s < NB)
    def _phase0():
        rows = pl.ds(i * B, B)
        adj_blk = jax.lax.dot_general(
            att_ref[...].astype(jnp.float8_e4m3fn), sp_ref[...],
            (((1,), (0,)), ((), ())),
            preferred_element_type=jnp.float32)
        adj_bf = adj_blk.astype(jnp.bfloat16)
        adj_scr[rows, :] = adj_bf

        @pl.when(i == 0)
        def _zero():
            t0_scr[...] = jnp.zeros_like(t0_scr)

        t0_scr[...] += jax.lax.dot_general(
            adj_bf, embs_ref[rows, :].astype(jnp.bfloat16),
            (((0,), (0,)), ((), ())),
            preferred_element_type=jnp.float32)

    @pl.when(s == NB)
    def _phase1():
        rows = pl.ds(0, C)
        adj_blk = adj_scr[rows, :]
        h = jnp.dot(adj_blk, t0_scr[...].astype(jnp.bfloat16),
                    preferred_element_type=jnp.float32)
        h = jnp.where(h >= 0, h, LEAKY * h)
        e1 = _ln(h, ln0w_ref[...], ln0b_ref[...]) + embs_ref[rows, :]

        t1_scr[...] = jnp.zeros_like(t1_scr)
        t1_scr[...] += jax.lax.dot_general(
            adj_blk, e1.astype(jnp.bfloat16), (((0,), (0,)), ((), ())),
            preferred_element_type=jnp.float32)

    @pl.when(s == NB + 1)
    def _phase2():
        rows = pl.ds(0, C)
        adj_blk = adj_scr[rows, :]
        h = jnp.dot(adj_blk, t1_scr[...].astype(jnp.bfloat16),
                    preferred_element_type=jnp.float32)
        out_ref[rows, :] = (_ln(h, ln1w_ref[...], ln1b_ref[...])
                            + embs_ref[rows, :])


def kernel(embs, sparse_adj, att_adj, ln0_w, ln0_b, ln1_w, ln1_b):
    grid = (NB + 2,)
    out = pl.pallas_call(
        _body,
        grid=grid,
        in_specs=[
            pl.BlockSpec((B, N), lambda s: (jnp.minimum(s, NB - 1), 0)),
            pl.BlockSpec((N, H), lambda s: (0, 0)),
            pl.BlockSpec((N, D), lambda s: (0, 0)),
            pl.BlockSpec((1, D), lambda s: (0, 0)),
            pl.BlockSpec((1, D), lambda s: (0, 0)),
            pl.BlockSpec((1, D), lambda s: (0, 0)),
            pl.BlockSpec((1, D), lambda s: (0, 0)),
        ],
        out_specs=pl.BlockSpec((N, D), lambda s: (0, 0)),
        out_shape=jax.ShapeDtypeStruct((N, D), jnp.float32),
        scratch_shapes=[
            pltpu.VMEM((N, H), jnp.bfloat16),
            pltpu.VMEM((H, D), jnp.float32),
            pltpu.VMEM((H, D), jnp.float32),
        ],
        compiler_params=pltpu.CompilerParams(
            dimension_semantics=("arbitrary",)),
    )(att_adj, sparse_adj.astype(jnp.float8_e4m3fn), embs,
      ln0_w.reshape(1, D), ln0_b.reshape(1, D),
      ln1_w.reshape(1, D), ln1_b.reshape(1, D))
    return out
